# baseline (speedup 1.0000x reference)
import functools

import jax
import jax.numpy as jnp
from jax import lax
from jax.experimental import pallas as pl
from jax.experimental.pallas import tpu as pltpu
from jax.experimental.pallas import tpu_sc as plsc

_N = 10000
_E = 320000
_D = 128

_NC = 2
_NS = 16
_NW = _NC * _NS
_CHUNK = 80
_NCHUNK = 128
_EP = _NW * _NCHUNK * _CHUNK
_NP = 10240
_RPT = _NP // _NS
_KBUF = 2
_IGB = 64

_mesh = plsc.VectorSubcoreMesh(core_axis_name="c", subcore_axis_name="s")


@functools.partial(
    pl.kernel,
    mesh=_mesh,
    out_type=jax.ShapeDtypeStruct((_NC, _NP, 16), jnp.float32),
    scratch_types=[
        pltpu.VMEM((_NCHUNK, _CHUNK), jnp.int32),
        pltpu.VMEM((_CHUNK, 16), jnp.float32),
        pltpu.VMEM_SHARED((_NP, 16), jnp.float32),
    ],
)
def _sc_degree(dstr_hbm, zeros16_hbm, out_hbm, dst_v, ones_v, acc_sh):
    c = lax.axis_index("c")
    s = lax.axis_index("s")
    wid = s * _NC + c

    pltpu.sync_copy(dstr_hbm.at[wid], dst_v)

    @pl.loop(0, _CHUNK)
    def _(i):
        ones_v[i, pl.ds(0, 16)] = jnp.ones((16,), jnp.float32)

    pltpu.sync_copy(
        zeros16_hbm.at[pl.ds(s * _RPT, _RPT)],
        acc_sh.at[pl.ds(s * _RPT, _RPT)],
    )
    plsc.subcore_barrier()

    @pl.loop(0, _NCHUNK)
    def _(j):
        pltpu.sync_copy(ones_v, acc_sh.at[dst_v.at[j]], add=True)

    plsc.subcore_barrier()
    pltpu.sync_copy(
        acc_sh.at[pl.ds(s * _RPT, _RPT)],
        out_hbm.at[c, pl.ds(s * _RPT, _RPT)],
    )


@functools.partial(
    pl.kernel,
    mesh=_mesh,
    out_type=jax.ShapeDtypeStruct((_NC, _NP, _D), jnp.float32),
    scratch_types=[
        pltpu.VMEM((_IGB, _CHUNK), jnp.int32),
        pltpu.VMEM((_IGB, _CHUNK), jnp.int32),
    ]
    + [pltpu.VMEM((_CHUNK, _D), jnp.float32) for _ in range(_KBUF)]
    + [pltpu.SemaphoreType.DMA for _ in range(_KBUF)]
    + [
        pltpu.VMEM_SHARED((_NP, _D), jnp.float32),
        pltpu.SemaphoreType.DMA,
    ],
)
def _sc_edges(g_hbm, srcr_hbm, dstr_hbm, zeros_hbm, out_hbm,
              src_v, dst_v, *rest):
    rows = rest[:_KBUF]
    gsems = rest[_KBUF:2 * _KBUF]
    acc_sh, ssem = rest[2 * _KBUF:]
    c = lax.axis_index("c")
    s = lax.axis_index("s")
    wid = s * _NC + c

    pltpu.sync_copy(
        zeros_hbm.at[pl.ds(s * _RPT, _RPT)],
        acc_sh.at[pl.ds(s * _RPT, _RPT)],
    )
    plsc.subcore_barrier()

    for og in range(_NCHUNK // _IGB):
        pltpu.sync_copy(srcr_hbm.at[wid, pl.ds(og * _IGB, _IGB)], src_v)
        pltpu.sync_copy(dstr_hbm.at[wid, pl.ds(og * _IGB, _IGB)], dst_v)

        @pl.loop(0, _IGB // _KBUF)
        def _(gidx):
            j0 = gidx * _KBUF
            ghs = [
                pltpu.async_copy(g_hbm.at[src_v.at[j0 + b]], rows[b], gsems[b])
                for b in range(_KBUF)
            ]
            for b in range(_KBUF):
                ghs[b].wait()
                pltpu.sync_copy(rows[b], acc_sh.at[dst_v.at[j0 + b]],
                                add=True)

    plsc.subcore_barrier()
    pltpu.sync_copy(
        acc_sh.at[pl.ds(s * _RPT, _RPT)],
        out_hbm.at[c, pl.ds(s * _RPT, _RPT)],
    )


_RB = 2048


def _dinv_block(degp_blk):
    deg = degp_blk[0, :, 0:1] + degp_blk[1, :, 0:1] + 1.0
    return lax.rsqrt(deg)


def _mm(x, W):
    def body(x_ref, w_ref, o_ref):
        o_ref[...] = jnp.dot(x_ref[...], w_ref[...],
                             preferred_element_type=jnp.float32)

    return pl.pallas_call(
        body,
        grid=(_NP // _RB,),
        in_specs=[
            pl.BlockSpec((_RB, _D), lambda i: (i, 0)),
            pl.BlockSpec((_D, _D), lambda i: (0, 0)),
        ],
        out_specs=pl.BlockSpec((_RB, _D), lambda i: (i, 0)),
        out_shape=jax.ShapeDtypeStruct((_NP, _D), jnp.float32),
    )(x, W)


def _scale(p, degp):
    def body(p_ref, degp_ref, o_ref):
        o_ref[...] = p_ref[...] * _dinv_block(degp_ref[...])

    return pl.pallas_call(
        body,
        grid=(_NP // _RB,),
        in_specs=[
            pl.BlockSpec((_RB, _D), lambda i: (i, 0)),
            pl.BlockSpec((_NC, _RB, 16), lambda i: (0, i, 0)),
        ],
        out_specs=pl.BlockSpec((_RB, _D), lambda i: (i, 0)),
        out_shape=jax.ShapeDtypeStruct((_NP, _D), jnp.float32),
    )(p, degp)


def _combine(acc, g, degp, b, W, bias_out, scale_out):
    def body(acc_ref, g_ref, degp_ref, b_ref, w_ref, bo_ref, o_ref):
        dinv = _dinv_block(degp_ref[...])
        tot = (acc_ref[0] + acc_ref[1] + g_ref[...]) * dinv + b_ref[...]
        h = jnp.maximum(tot, 0.0)
        o = jnp.dot(h, w_ref[...], preferred_element_type=jnp.float32)
        if scale_out:
            o = o * dinv
        else:
            o = o + bo_ref[...]
        o_ref[...] = o

    bo = bias_out if bias_out is not None else jnp.zeros((_D,), jnp.float32)
    return pl.pallas_call(
        body,
        grid=(_NP // _RB,),
        in_specs=[
            pl.BlockSpec((_NC, _RB, _D), lambda i: (0, i, 0)),
            pl.BlockSpec((_RB, _D), lambda i: (i, 0)),
            pl.BlockSpec((_NC, _RB, 16), lambda i: (0, i, 0)),
            pl.BlockSpec((1, _D), lambda i: (0, 0)),
            pl.BlockSpec((_D, _D), lambda i: (0, 0)),
            pl.BlockSpec((1, _D), lambda i: (0, 0)),
        ],
        out_specs=pl.BlockSpec((_RB, _D), lambda i: (i, 0)),
        out_shape=jax.ShapeDtypeStruct((_NP, _D), jnp.float32),
    )(acc, g, degp, b.reshape(1, _D), W, bo.reshape(1, _D))


def kernel(x, edge_index, W1, b1, W2, b2, Wl, bl):
    pad = jnp.full((2, _EP - _E), _NP - 1, jnp.int32)
    ei = jnp.concatenate([edge_index, pad], axis=1)
    srcr = ei[0].reshape(_NW, _NCHUNK, _CHUNK)
    dstr = ei[1].reshape(_NW, _NCHUNK, _CHUNK)
    xp = jnp.zeros((_NP, _D), jnp.float32).at[:_N].set(x)
    zeros = jnp.zeros((_NP, _D), jnp.float32)
    zeros16 = jnp.zeros((_NP, 16), jnp.float32)

    degp = _sc_degree(dstr, zeros16)
    p1 = _mm(xp, W1)
    g1 = _scale(p1, degp)
    acc1 = _sc_edges(g1, srcr, dstr, zeros)
    g2 = _combine(acc1, g1, degp, b1, W2, None, scale_out=True)
    acc2 = _sc_edges(g2, srcr, dstr, zeros)
    out = _combine(acc2, g2, degp, b2, Wl, bl, scale_out=False)
    return out[:_N]


# speedup vs baseline: 2.2434x; 2.2434x over previous
import functools

import jax
import jax.numpy as jnp
from jax import lax
from jax.experimental import pallas as pl
from jax.experimental.pallas import tpu as pltpu
from jax.experimental.pallas import tpu_sc as plsc

_N = 10000
_E = 320000
_D = 128

_NC = 2
_NS = 16
_NW = _NC * _NS
_CHUNK = 80
_NCHUNK = _E // (_NW * _CHUNK)
_NP = 10240
_RPT = _NP // _NS

_mesh = plsc.VectorSubcoreMesh(core_axis_name="c", subcore_axis_name="s")


@functools.partial(
    pl.kernel,
    mesh=_mesh,
    out_type=jax.ShapeDtypeStruct((_NC, _NP, 16), jnp.float32),
    scratch_types=[
        pltpu.VMEM((_NCHUNK, _CHUNK), jnp.int32),
        pltpu.VMEM((_CHUNK, 16), jnp.float32),
        pltpu.VMEM_SHARED((_NP, 16), jnp.float32),
    ],
)
def _sc_degree(dstr_hbm, zeros16_hbm, out_hbm, dst_v, ones_v, acc_sh):
    c = lax.axis_index("c")
    s = lax.axis_index("s")
    wid = s * _NC + c

    pltpu.sync_copy(dstr_hbm.at[wid], dst_v)

    @pl.loop(0, _CHUNK)
    def _(i):
        ones_v[i, pl.ds(0, 16)] = jnp.ones((16,), jnp.float32)

    pltpu.sync_copy(
        zeros16_hbm.at[pl.ds(s * _RPT, _RPT)],
        acc_sh.at[pl.ds(s * _RPT, _RPT)],
    )
    plsc.subcore_barrier()

    @pl.loop(0, _NCHUNK)
    def _(j):
        pltpu.sync_copy(ones_v, acc_sh.at[dst_v.at[j]], add=True)

    plsc.subcore_barrier()
    pltpu.sync_copy(
        acc_sh.at[pl.ds(s * _RPT, _RPT)],
        out_hbm.at[c, pl.ds(s * _RPT, _RPT)],
    )


@functools.partial(
    pl.kernel,
    mesh=_mesh,
    out_type=jax.ShapeDtypeStruct((_NC, _NP, _D), jnp.float32),
    scratch_types=[
        pltpu.VMEM((_NCHUNK, _CHUNK), jnp.int32),
        pltpu.VMEM((_NCHUNK, _CHUNK), jnp.int32),
        pltpu.VMEM((_CHUNK, _D), jnp.float32),
        pltpu.VMEM_SHARED((_NP, _D), jnp.float32),
        pltpu.SemaphoreType.DMA,
    ],
)
def _sc_edges(g_hbm, srcr_hbm, dstr_hbm, zeros_hbm, out_hbm,
              src_v, dst_v, rows_v, acc_sh, sem):
    c = lax.axis_index("c")
    s = lax.axis_index("s")
    wid = s * _NC + c

    pltpu.sync_copy(srcr_hbm.at[wid], src_v)
    pltpu.sync_copy(dstr_hbm.at[wid], dst_v)

    pltpu.sync_copy(
        zeros_hbm.at[pl.ds(s * _RPT, _RPT)],
        acc_sh.at[pl.ds(s * _RPT, _RPT)],
    )
    plsc.subcore_barrier()

    @pl.loop(0, _NCHUNK)
    def _(j):
        pltpu.async_copy(g_hbm.at[src_v.at[j]], rows_v, sem).wait()
        pltpu.sync_copy(rows_v, acc_sh.at[dst_v.at[j]], add=True)

    plsc.subcore_barrier()
    pltpu.sync_copy(
        acc_sh.at[pl.ds(s * _RPT, _RPT)],
        out_hbm.at[c, pl.ds(s * _RPT, _RPT)],
    )


_RB = 2048


def _dinv_block(degp_blk):
    deg = degp_blk[0, :, 0:1] + degp_blk[1, :, 0:1] + 1.0
    return lax.rsqrt(deg)


def _mm_scale(x, W, degp):
    def body(x_ref, w_ref, degp_ref, o_ref):
        dinv = _dinv_block(degp_ref[...])
        h = jnp.dot(x_ref[...], w_ref[...], preferred_element_type=jnp.float32)
        o_ref[...] = h * dinv

    return pl.pallas_call(
        body,
        grid=(_NP // _RB,),
        in_specs=[
            pl.BlockSpec((_RB, _D), lambda i: (i, 0)),
            pl.BlockSpec((_D, _D), lambda i: (0, 0)),
            pl.BlockSpec((_NC, _RB, 16), lambda i: (0, i, 0)),
        ],
        out_specs=pl.BlockSpec((_RB, _D), lambda i: (i, 0)),
        out_shape=jax.ShapeDtypeStruct((_NP, _D), jnp.float32),
    )(x, W, degp)


def _combine(acc, g, degp, b, W, bias_out, scale_out):
    def body(acc_ref, g_ref, degp_ref, b_ref, w_ref, bo_ref, o_ref):
        dinv = _dinv_block(degp_ref[...])
        tot = (acc_ref[0] + acc_ref[1] + g_ref[...]) * dinv + b_ref[...]
        h = jnp.maximum(tot, 0.0)
        o = jnp.dot(h, w_ref[...], preferred_element_type=jnp.float32)
        if scale_out:
            o = o * dinv
        else:
            o = o + bo_ref[...]
        o_ref[...] = o

    bo = bias_out if bias_out is not None else jnp.zeros((_D,), jnp.float32)
    return pl.pallas_call(
        body,
        grid=(_NP // _RB,),
        in_specs=[
            pl.BlockSpec((_NC, _RB, _D), lambda i: (0, i, 0)),
            pl.BlockSpec((_RB, _D), lambda i: (i, 0)),
            pl.BlockSpec((_NC, _RB, 16), lambda i: (0, i, 0)),
            pl.BlockSpec((1, _D), lambda i: (0, 0)),
            pl.BlockSpec((_D, _D), lambda i: (0, 0)),
            pl.BlockSpec((1, _D), lambda i: (0, 0)),
        ],
        out_specs=pl.BlockSpec((_RB, _D), lambda i: (i, 0)),
        out_shape=jax.ShapeDtypeStruct((_NP, _D), jnp.float32),
    )(acc, g, degp, b.reshape(1, _D), W, bo.reshape(1, _D))


def kernel(x, edge_index, W1, b1, W2, b2, Wl, bl):
    srcr = edge_index[0].reshape(_NW, _NCHUNK, _CHUNK)
    dstr = edge_index[1].reshape(_NW, _NCHUNK, _CHUNK)
    xp = jnp.zeros((_NP, _D), jnp.float32).at[:_N].set(x)
    zeros = jnp.zeros((_NP, _D), jnp.float32)
    zeros16 = jnp.zeros((_NP, 16), jnp.float32)

    degp = _sc_degree(dstr, zeros16)
    g1 = _mm_scale(xp, W1, degp)
    acc1 = _sc_edges(g1, srcr, dstr, zeros)
    g2 = _combine(acc1, g1, degp, b1, W2, None, scale_out=True)
    acc2 = _sc_edges(g2, srcr, dstr, zeros)
    out = _combine(acc2, g2, degp, b2, Wl, bl, scale_out=False)
    return out[:_N]


# speedup vs baseline: 2.6004x; 1.1592x over previous
import functools

import jax
import jax.numpy as jnp
from jax import lax
from jax.experimental import pallas as pl
from jax.experimental.pallas import tpu as pltpu
from jax.experimental.pallas import tpu_sc as plsc

_N = 10000
_E = 320000
_D = 128

_NC = 2
_NS = 16
_NW = _NC * _NS
_CHUNK = 80
_NCHUNK = 128
_EP = _NW * _NCHUNK * _CHUNK
_NPAD = _EP - _E
_KBUF = 2
_IGB = 64
_NP = 10240
_RPT = _NP // _NS

_mesh = plsc.VectorSubcoreMesh(core_axis_name="c", subcore_axis_name="s")


@functools.partial(
    pl.kernel,
    mesh=_mesh,
    out_type=jax.ShapeDtypeStruct((_NC, _NP, 16), jnp.float32),
    scratch_types=[
        pltpu.VMEM((_NCHUNK, _CHUNK), jnp.int32),
        pltpu.VMEM((_CHUNK, 16), jnp.float32),
        pltpu.VMEM_SHARED((_NP, 16), jnp.float32),
    ],
)
def _sc_degree(dstr_hbm, zeros16_hbm, out_hbm, dst_v, ones_v, acc_sh):
    c = lax.axis_index("c")
    s = lax.axis_index("s")
    wid = s * _NC + c

    pltpu.sync_copy(dstr_hbm.at[wid], dst_v)

    @pl.loop(0, _CHUNK)
    def _(i):
        ones_v[i, pl.ds(0, 16)] = jnp.ones((16,), jnp.float32)

    pltpu.sync_copy(
        zeros16_hbm.at[pl.ds(s * _RPT, _RPT)],
        acc_sh.at[pl.ds(s * _RPT, _RPT)],
    )
    plsc.subcore_barrier()

    @pl.loop(0, _NCHUNK)
    def _(j):
        pltpu.sync_copy(ones_v, acc_sh.at[dst_v.at[j]], add=True)

    plsc.subcore_barrier()
    pltpu.sync_copy(
        acc_sh.at[pl.ds(s * _RPT, _RPT)],
        out_hbm.at[c, pl.ds(s * _RPT, _RPT)],
    )


@functools.partial(
    pl.kernel,
    mesh=_mesh,
    out_type=jax.ShapeDtypeStruct((_NC, _NP, _D), jnp.float32),
    scratch_types=[
        pltpu.VMEM((_IGB, _CHUNK), jnp.int32),
        pltpu.VMEM((_IGB, _CHUNK), jnp.int32),
        pltpu.VMEM((_CHUNK, _D), jnp.float32),
        pltpu.VMEM((_CHUNK, _D), jnp.float32),
        pltpu.VMEM_SHARED((_NP, _D), jnp.float32),
        pltpu.SemaphoreType.DMA,
        pltpu.SemaphoreType.DMA,
    ],
)
def _sc_edges(g_hbm, srcr_hbm, dstr_hbm, zeros_hbm, out_hbm,
              src_v, dst_v, rows0_v, rows1_v, acc_sh, sem0, sem1):
    c = lax.axis_index("c")
    s = lax.axis_index("s")
    wid = s * _NC + c

    pltpu.sync_copy(
        zeros_hbm.at[pl.ds(s * _RPT, _RPT)],
        acc_sh.at[pl.ds(s * _RPT, _RPT)],
    )
    plsc.subcore_barrier()

    for og in range(_NCHUNK // _IGB):
        pltpu.sync_copy(srcr_hbm.at[wid, pl.ds(og * _IGB, _IGB)], src_v)
        pltpu.sync_copy(dstr_hbm.at[wid, pl.ds(og * _IGB, _IGB)], dst_v)

        @pl.loop(0, _IGB // 2)
        def _(gidx):
            j0 = gidx * 2
            h0 = pltpu.async_copy(g_hbm.at[src_v.at[j0]], rows0_v, sem0)
            h1 = pltpu.async_copy(g_hbm.at[src_v.at[j0 + 1]], rows1_v, sem1)
            h0.wait()
            h1.wait()
            pltpu.sync_copy(rows0_v, acc_sh.at[dst_v.at[j0]], add=True)
            pltpu.sync_copy(rows1_v, acc_sh.at[dst_v.at[j0 + 1]], add=True)

    plsc.subcore_barrier()
    pltpu.sync_copy(
        acc_sh.at[pl.ds(s * _RPT, _RPT)],
        out_hbm.at[c, pl.ds(s * _RPT, _RPT)],
    )


_RB = 2048


def _dinv_block(degp_blk):
    deg = degp_blk[0, :, 0:1] + degp_blk[1, :, 0:1] + 1.0
    return lax.rsqrt(deg)


def _mm_scale(x, W, degp):
    def body(x_ref, w_ref, degp_ref, o_ref):
        dinv = _dinv_block(degp_ref[...])
        h = jnp.dot(x_ref[...], w_ref[...], preferred_element_type=jnp.float32)
        o_ref[...] = h * dinv

    return pl.pallas_call(
        body,
        grid=(_NP // _RB,),
        in_specs=[
            pl.BlockSpec((_RB, _D), lambda i: (i, 0)),
            pl.BlockSpec((_D, _D), lambda i: (0, 0)),
            pl.BlockSpec((_NC, _RB, 16), lambda i: (0, i, 0)),
        ],
        out_specs=pl.BlockSpec((_RB, _D), lambda i: (i, 0)),
        out_shape=jax.ShapeDtypeStruct((_NP, _D), jnp.float32),
    )(x, W, degp)


def _combine(acc, g, degp, b, W, bias_out, scale_out):
    def body(acc_ref, g_ref, degp_ref, b_ref, w_ref, bo_ref, o_ref):
        dinv = _dinv_block(degp_ref[...])
        tot = (acc_ref[0] + acc_ref[1] + g_ref[...]) * dinv + b_ref[...]
        h = jnp.maximum(tot, 0.0)
        o = jnp.dot(h, w_ref[...], preferred_element_type=jnp.float32)
        if scale_out:
            o = o * dinv
        else:
            o = o + bo_ref[...]
        o_ref[...] = o

    bo = bias_out if bias_out is not None else jnp.zeros((_D,), jnp.float32)
    return pl.pallas_call(
        body,
        grid=(_NP // _RB,),
        in_specs=[
            pl.BlockSpec((_NC, _RB, _D), lambda i: (0, i, 0)),
            pl.BlockSpec((_RB, _D), lambda i: (i, 0)),
            pl.BlockSpec((_NC, _RB, 16), lambda i: (0, i, 0)),
            pl.BlockSpec((1, _D), lambda i: (0, 0)),
            pl.BlockSpec((_D, _D), lambda i: (0, 0)),
            pl.BlockSpec((1, _D), lambda i: (0, 0)),
        ],
        out_specs=pl.BlockSpec((_RB, _D), lambda i: (i, 0)),
        out_shape=jax.ShapeDtypeStruct((_NP, _D), jnp.float32),
    )(acc, g, degp, b.reshape(1, _D), W, bo.reshape(1, _D))


def kernel(x, edge_index, W1, b1, W2, b2, Wl, bl):
    padv = _N + (jnp.arange(_NPAD, dtype=jnp.int32) % (_NP - _N))
    pad = jnp.broadcast_to(padv, (2, _NPAD))
    ei = jnp.concatenate([edge_index, pad], axis=1)
    srcr = ei[0].reshape(_NW, _NCHUNK, _CHUNK)
    dstr = ei[1].reshape(_NW, _NCHUNK, _CHUNK)
    xp = jnp.zeros((_NP, _D), jnp.float32).at[:_N].set(x)
    zeros = jnp.zeros((_NP, _D), jnp.float32)
    zeros16 = jnp.zeros((_NP, 16), jnp.float32)

    degp = _sc_degree(dstr, zeros16)
    g1 = _mm_scale(xp, W1, degp)
    acc1 = _sc_edges(g1, srcr, dstr, zeros)
    g2 = _combine(acc1, g1, degp, b1, W2, None, scale_out=True)
    acc2 = _sc_edges(g2, srcr, dstr, zeros)
    out = _combine(acc2, g2, degp, b2, Wl, bl, scale_out=False)
    return out[:_N]


# speedup vs baseline: 2.6435x; 1.0166x over previous
import functools

import jax
import jax.numpy as jnp
from jax import lax
from jax.experimental import pallas as pl
from jax.experimental.pallas import tpu as pltpu
from jax.experimental.pallas import tpu_sc as plsc

_N = 10000
_E = 320000
_D = 128

_NC = 2
_NS = 16
_NW = _NC * _NS
_CHUNK = 80
_NCHUNK = 128
_EP = _NW * _NCHUNK * _CHUNK
_NPAD = _EP - _E
_KBUF = 2
_IGB = 64
_NP = 10240
_RPT = _NP // _NS

_mesh = plsc.VectorSubcoreMesh(core_axis_name="c", subcore_axis_name="s")


@functools.partial(
    pl.kernel,
    mesh=_mesh,
    out_type=jax.ShapeDtypeStruct((_NC, _NP, 16), jnp.float32),
    scratch_types=[
        pltpu.VMEM((_NCHUNK, _CHUNK), jnp.int32),
        pltpu.VMEM((_CHUNK, 16), jnp.float32),
        pltpu.VMEM_SHARED((_NP, 16), jnp.float32),
    ],
)
def _sc_degree(dstr_hbm, zeros16_hbm, out_hbm, dst_v, ones_v, acc_sh):
    c = lax.axis_index("c")
    s = lax.axis_index("s")
    wid = s * _NC + c

    pltpu.sync_copy(dstr_hbm.at[wid], dst_v)

    @pl.loop(0, _CHUNK)
    def _(i):
        ones_v[i, pl.ds(0, 16)] = jnp.ones((16,), jnp.float32)

    pltpu.sync_copy(
        zeros16_hbm.at[pl.ds(s * _RPT, _RPT)],
        acc_sh.at[pl.ds(s * _RPT, _RPT)],
    )
    plsc.subcore_barrier()

    @pl.loop(0, _NCHUNK)
    def _(j):
        pltpu.sync_copy(ones_v, acc_sh.at[dst_v.at[j]], add=True)

    plsc.subcore_barrier()
    pltpu.sync_copy(
        acc_sh.at[pl.ds(s * _RPT, _RPT)],
        out_hbm.at[c, pl.ds(s * _RPT, _RPT)],
    )


@functools.partial(
    pl.kernel,
    mesh=_mesh,
    out_type=jax.ShapeDtypeStruct((_NC, _NP, _D), jnp.float32),
    scratch_types=[
        pltpu.VMEM((_IGB, _CHUNK), jnp.int32),
        pltpu.VMEM((_IGB, _CHUNK), jnp.int32),
        pltpu.VMEM((_CHUNK, _D), jnp.float32),
        pltpu.VMEM((_CHUNK, _D), jnp.float32),
        pltpu.VMEM_SHARED((_NP, _D), jnp.float32),
        pltpu.SemaphoreType.DMA,
        pltpu.SemaphoreType.DMA,
    ],
)
def _sc_edges(g_hbm, srcr_hbm, dstr_hbm, zeros_hbm, out_hbm,
              src_v, dst_v, rows0_v, rows1_v, acc_sh, sem0, sem1):
    c = lax.axis_index("c")
    s = lax.axis_index("s")
    wid = s * _NC + c

    pltpu.sync_copy(
        zeros_hbm.at[pl.ds(s * _RPT, _RPT)],
        acc_sh.at[pl.ds(s * _RPT, _RPT)],
    )
    plsc.subcore_barrier()

    for og in range(_NCHUNK // _IGB):
        pltpu.sync_copy(srcr_hbm.at[wid, pl.ds(og * _IGB, _IGB)], src_v)
        pltpu.sync_copy(dstr_hbm.at[wid, pl.ds(og * _IGB, _IGB)], dst_v)

        @pl.loop(0, _IGB // 2)
        def _(gidx):
            j0 = gidx * 2
            h0 = pltpu.async_copy(g_hbm.at[src_v.at[j0]], rows0_v, sem0)
            h1 = pltpu.async_copy(g_hbm.at[src_v.at[j0 + 1]], rows1_v, sem1)
            h0.wait()
            h1.wait()
            s0 = pltpu.async_copy(rows0_v, acc_sh.at[dst_v.at[j0]], sem0,
                                  add=True)
            s1 = pltpu.async_copy(rows1_v, acc_sh.at[dst_v.at[j0 + 1]], sem1,
                                  add=True)
            s0.wait()
            s1.wait()

    plsc.subcore_barrier()
    pltpu.sync_copy(
        acc_sh.at[pl.ds(s * _RPT, _RPT)],
        out_hbm.at[c, pl.ds(s * _RPT, _RPT)],
    )


_RB = 2048


def _dinv_block(degp_blk):
    deg = degp_blk[0, :, 0:1] + degp_blk[1, :, 0:1] + 1.0
    return lax.rsqrt(deg)


def _mm_scale(x, W, degp):
    def body(x_ref, w_ref, degp_ref, o_ref):
        dinv = _dinv_block(degp_ref[...])
        h = jnp.dot(x_ref[...], w_ref[...], preferred_element_type=jnp.float32)
        o_ref[...] = h * dinv

    return pl.pallas_call(
        body,
        grid=(_NP // _RB,),
        in_specs=[
            pl.BlockSpec((_RB, _D), lambda i: (i, 0)),
            pl.BlockSpec((_D, _D), lambda i: (0, 0)),
            pl.BlockSpec((_NC, _RB, 16), lambda i: (0, i, 0)),
        ],
        out_specs=pl.BlockSpec((_RB, _D), lambda i: (i, 0)),
        out_shape=jax.ShapeDtypeStruct((_NP, _D), jnp.float32),
    )(x, W, degp)


def _combine(acc, g, degp, b, W, bias_out, scale_out):
    def body(acc_ref, g_ref, degp_ref, b_ref, w_ref, bo_ref, o_ref):
        dinv = _dinv_block(degp_ref[...])
        tot = (acc_ref[0] + acc_ref[1] + g_ref[...]) * dinv + b_ref[...]
        h = jnp.maximum(tot, 0.0)
        o = jnp.dot(h, w_ref[...], preferred_element_type=jnp.float32)
        if scale_out:
            o = o * dinv
        else:
            o = o + bo_ref[...]
        o_ref[...] = o

    bo = bias_out if bias_out is not None else jnp.zeros((_D,), jnp.float32)
    return pl.pallas_call(
        body,
        grid=(_NP // _RB,),
        in_specs=[
            pl.BlockSpec((_NC, _RB, _D), lambda i: (0, i, 0)),
            pl.BlockSpec((_RB, _D), lambda i: (i, 0)),
            pl.BlockSpec((_NC, _RB, 16), lambda i: (0, i, 0)),
            pl.BlockSpec((1, _D), lambda i: (0, 0)),
            pl.BlockSpec((_D, _D), lambda i: (0, 0)),
            pl.BlockSpec((1, _D), lambda i: (0, 0)),
        ],
        out_specs=pl.BlockSpec((_RB, _D), lambda i: (i, 0)),
        out_shape=jax.ShapeDtypeStruct((_NP, _D), jnp.float32),
    )(acc, g, degp, b.reshape(1, _D), W, bo.reshape(1, _D))


def kernel(x, edge_index, W1, b1, W2, b2, Wl, bl):
    padv = _N + (jnp.arange(_NPAD, dtype=jnp.int32) % (_NP - _N))
    pad = jnp.broadcast_to(padv, (2, _NPAD))
    ei = jnp.concatenate([edge_index, pad], axis=1)
    srcr = ei[0].reshape(_NW, _NCHUNK, _CHUNK)
    dstr = ei[1].reshape(_NW, _NCHUNK, _CHUNK)
    xp = jnp.zeros((_NP, _D), jnp.float32).at[:_N].set(x)
    zeros = jnp.zeros((_NP, _D), jnp.float32)
    zeros16 = jnp.zeros((_NP, 16), jnp.float32)

    degp = _sc_degree(dstr, zeros16)
    g1 = _mm_scale(xp, W1, degp)
    acc1 = _sc_edges(g1, srcr, dstr, zeros)
    g2 = _combine(acc1, g1, degp, b1, W2, None, scale_out=True)
    acc2 = _sc_edges(g2, srcr, dstr, zeros)
    out = _combine(acc2, g2, degp, b2, Wl, bl, scale_out=False)
    return out[:_N]


# speedup vs baseline: 2.7843x; 1.0533x over previous
import functools

import jax
import jax.numpy as jnp
from jax import lax
from jax.experimental import pallas as pl
from jax.experimental.pallas import tpu as pltpu
from jax.experimental.pallas import tpu_sc as plsc

_N = 10000
_E = 320000
_D = 128

_NC = 2
_NS = 16
_NW = _NC * _NS
_CHUNK = 128
_NCHUNK = 80
_EP = _NW * _NCHUNK * _CHUNK
_NPAD = _EP - _E
_KBUF = 2
_IGB = 16
_NP = 10240
_RPT = _NP // _NS

_mesh = plsc.VectorSubcoreMesh(core_axis_name="c", subcore_axis_name="s")


@functools.partial(
    pl.kernel,
    mesh=_mesh,
    out_type=jax.ShapeDtypeStruct((_NC, _NP, 16), jnp.float32),
    scratch_types=[
        pltpu.VMEM((_NCHUNK, _CHUNK), jnp.int32),
        pltpu.VMEM((_CHUNK, 16), jnp.float32),
        pltpu.VMEM_SHARED((_NP, 16), jnp.float32),
    ],
)
def _sc_degree(dstr_hbm, zeros16_hbm, out_hbm, dst_v, ones_v, acc_sh):
    c = lax.axis_index("c")
    s = lax.axis_index("s")
    wid = s * _NC + c

    pltpu.sync_copy(dstr_hbm.at[wid], dst_v)

    @pl.loop(0, _CHUNK)
    def _(i):
        ones_v[i, pl.ds(0, 16)] = jnp.ones((16,), jnp.float32)

    pltpu.sync_copy(
        zeros16_hbm.at[pl.ds(s * _RPT, _RPT)],
        acc_sh.at[pl.ds(s * _RPT, _RPT)],
    )
    plsc.subcore_barrier()

    @pl.loop(0, _NCHUNK)
    def _(j):
        pltpu.sync_copy(ones_v, acc_sh.at[dst_v.at[j]], add=True)

    plsc.subcore_barrier()
    pltpu.sync_copy(
        acc_sh.at[pl.ds(s * _RPT, _RPT)],
        out_hbm.at[c, pl.ds(s * _RPT, _RPT)],
    )


@functools.partial(
    pl.kernel,
    mesh=_mesh,
    out_type=jax.ShapeDtypeStruct((_NC, _NP, _D), jnp.float32),
    scratch_types=[
        pltpu.VMEM((_IGB, _CHUNK), jnp.int32),
        pltpu.VMEM((_IGB, _CHUNK), jnp.int32),
        pltpu.VMEM((_CHUNK, _D), jnp.float32),
        pltpu.VMEM((_CHUNK, _D), jnp.float32),
        pltpu.VMEM_SHARED((_NP, _D), jnp.float32),
        pltpu.SemaphoreType.DMA,
        pltpu.SemaphoreType.DMA,
    ],
)
def _sc_edges(g_hbm, srcr_hbm, dstr_hbm, zeros_hbm, out_hbm,
              src_v, dst_v, rows0_v, rows1_v, acc_sh, sem0, sem1):
    c = lax.axis_index("c")
    s = lax.axis_index("s")
    wid = s * _NC + c

    pltpu.sync_copy(
        zeros_hbm.at[pl.ds(s * _RPT, _RPT)],
        acc_sh.at[pl.ds(s * _RPT, _RPT)],
    )
    plsc.subcore_barrier()

    for og in range(_NCHUNK // _IGB):
        pltpu.sync_copy(srcr_hbm.at[wid, pl.ds(og * _IGB, _IGB)], src_v)
        pltpu.sync_copy(dstr_hbm.at[wid, pl.ds(og * _IGB, _IGB)], dst_v)

        @pl.loop(0, _IGB // 2)
        def _(gidx):
            j0 = gidx * 2
            h0 = pltpu.async_copy(g_hbm.at[src_v.at[j0]], rows0_v, sem0)
            h1 = pltpu.async_copy(g_hbm.at[src_v.at[j0 + 1]], rows1_v, sem1)
            h0.wait()
            h1.wait()
            pltpu.sync_copy(rows0_v, acc_sh.at[dst_v.at[j0]], add=True)
            pltpu.sync_copy(rows1_v, acc_sh.at[dst_v.at[j0 + 1]], add=True)

    plsc.subcore_barrier()
    pltpu.sync_copy(
        acc_sh.at[pl.ds(s * _RPT, _RPT)],
        out_hbm.at[c, pl.ds(s * _RPT, _RPT)],
    )


_RB = 2048


def _dinv_block(degp_blk):
    deg = degp_blk[0, :, 0:1] + degp_blk[1, :, 0:1] + 1.0
    return lax.rsqrt(deg)


def _mm_scale(x, W, degp):
    def body(x_ref, w_ref, degp_ref, o_ref):
        dinv = _dinv_block(degp_ref[...])
        h = jnp.dot(x_ref[...], w_ref[...], preferred_element_type=jnp.float32)
        o_ref[...] = h * dinv

    return pl.pallas_call(
        body,
        grid=(_NP // _RB,),
        in_specs=[
            pl.BlockSpec((_RB, _D), lambda i: (i, 0)),
            pl.BlockSpec((_D, _D), lambda i: (0, 0)),
            pl.BlockSpec((_NC, _RB, 16), lambda i: (0, i, 0)),
        ],
        out_specs=pl.BlockSpec((_RB, _D), lambda i: (i, 0)),
        out_shape=jax.ShapeDtypeStruct((_NP, _D), jnp.float32),
    )(x, W, degp)


def _combine(acc, g, degp, b, W, bias_out, scale_out):
    def body(acc_ref, g_ref, degp_ref, b_ref, w_ref, bo_ref, o_ref):
        dinv = _dinv_block(degp_ref[...])
        tot = (acc_ref[0] + acc_ref[1] + g_ref[...]) * dinv + b_ref[...]
        h = jnp.maximum(tot, 0.0)
        o = jnp.dot(h, w_ref[...], preferred_element_type=jnp.float32)
        if scale_out:
            o = o * dinv
        else:
            o = o + bo_ref[...]
        o_ref[...] = o

    bo = bias_out if bias_out is not None else jnp.zeros((_D,), jnp.float32)
    return pl.pallas_call(
        body,
        grid=(_NP // _RB,),
        in_specs=[
            pl.BlockSpec((_NC, _RB, _D), lambda i: (0, i, 0)),
            pl.BlockSpec((_RB, _D), lambda i: (i, 0)),
            pl.BlockSpec((_NC, _RB, 16), lambda i: (0, i, 0)),
            pl.BlockSpec((1, _D), lambda i: (0, 0)),
            pl.BlockSpec((_D, _D), lambda i: (0, 0)),
            pl.BlockSpec((1, _D), lambda i: (0, 0)),
        ],
        out_specs=pl.BlockSpec((_RB, _D), lambda i: (i, 0)),
        out_shape=jax.ShapeDtypeStruct((_NP, _D), jnp.float32),
    )(acc, g, degp, b.reshape(1, _D), W, bo.reshape(1, _D))


def kernel(x, edge_index, W1, b1, W2, b2, Wl, bl):
    padv = _N + (jnp.arange(_NPAD, dtype=jnp.int32) % (_NP - _N))
    pad = jnp.broadcast_to(padv, (2, _NPAD))
    ei = jnp.concatenate([edge_index, pad], axis=1)
    srcr = ei[0].reshape(_NW, _NCHUNK, _CHUNK)
    dstr = ei[1].reshape(_NW, _NCHUNK, _CHUNK)
    xp = jnp.zeros((_NP, _D), jnp.float32).at[:_N].set(x)
    zeros = jnp.zeros((_NP, _D), jnp.float32)
    zeros16 = jnp.zeros((_NP, 16), jnp.float32)

    degp = _sc_degree(dstr, zeros16)
    g1 = _mm_scale(xp, W1, degp)
    acc1 = _sc_edges(g1, srcr, dstr, zeros)
    g2 = _combine(acc1, g1, degp, b1, W2, None, scale_out=True)
    acc2 = _sc_edges(g2, srcr, dstr, zeros)
    out = _combine(acc2, g2, degp, b2, Wl, bl, scale_out=False)
    return out[:_N]


# speedup vs baseline: 2.8605x; 1.0274x over previous
import functools

import jax
import jax.numpy as jnp
from jax import lax
from jax.experimental import pallas as pl
from jax.experimental.pallas import tpu as pltpu
from jax.experimental.pallas import tpu_sc as plsc

_N = 10000
_E = 320000
_D = 128

_NC = 2
_NS = 16
_NW = _NC * _NS
_CHUNK = 128
_NCHUNK = 80
_EP = _NW * _NCHUNK * _CHUNK
_NPAD = _EP - _E
_KBUF = 2
_IGB = 40
_NP = 10240
_RPT = _NP // _NS

_mesh = plsc.VectorSubcoreMesh(core_axis_name="c", subcore_axis_name="s")


@functools.partial(
    pl.kernel,
    mesh=_mesh,
    out_type=jax.ShapeDtypeStruct((_NC, _NP, 16), jnp.float32),
    scratch_types=[
        pltpu.VMEM((_NCHUNK, _CHUNK), jnp.int32),
        pltpu.VMEM((_CHUNK, 16), jnp.float32),
        pltpu.VMEM_SHARED((_NP, 16), jnp.float32),
    ],
)
def _sc_degree(dstr_hbm, zeros16_hbm, out_hbm, dst_v, ones_v, acc_sh):
    c = lax.axis_index("c")
    s = lax.axis_index("s")
    wid = s * _NC + c

    pltpu.sync_copy(dstr_hbm.at[wid], dst_v)

    @pl.loop(0, _CHUNK)
    def _(i):
        ones_v[i, pl.ds(0, 16)] = jnp.ones((16,), jnp.float32)

    pltpu.sync_copy(
        zeros16_hbm.at[pl.ds(s * _RPT, _RPT)],
        acc_sh.at[pl.ds(s * _RPT, _RPT)],
    )
    plsc.subcore_barrier()

    @pl.loop(0, _NCHUNK)
    def _(j):
        pltpu.sync_copy(ones_v, acc_sh.at[dst_v.at[j]], add=True)

    plsc.subcore_barrier()
    pltpu.sync_copy(
        acc_sh.at[pl.ds(s * _RPT, _RPT)],
        out_hbm.at[c, pl.ds(s * _RPT, _RPT)],
    )


@functools.partial(
    pl.kernel,
    mesh=_mesh,
    out_type=jax.ShapeDtypeStruct((_NC, _NP, _D), jnp.float32),
    scratch_types=[
        pltpu.VMEM((_IGB, _CHUNK), jnp.int32),
        pltpu.VMEM((_IGB, _CHUNK), jnp.int32),
        pltpu.VMEM((_CHUNK, _D), jnp.float32),
        pltpu.VMEM((_CHUNK, _D), jnp.float32),
        pltpu.VMEM_SHARED((_NP, _D), jnp.float32),
        pltpu.SemaphoreType.DMA,
        pltpu.SemaphoreType.DMA,
    ],
)
def _sc_edges(g_hbm, srcr_hbm, dstr_hbm, zeros_hbm, out_hbm,
              src_v, dst_v, rows0_v, rows1_v, acc_sh, sem0, sem1):
    c = lax.axis_index("c")
    s = lax.axis_index("s")
    wid = s * _NC + c

    pltpu.sync_copy(
        zeros_hbm.at[pl.ds(s * _RPT, _RPT)],
        acc_sh.at[pl.ds(s * _RPT, _RPT)],
    )
    plsc.subcore_barrier()

    for og in range(_NCHUNK // _IGB):
        pltpu.sync_copy(srcr_hbm.at[wid, pl.ds(og * _IGB, _IGB)], src_v)
        pltpu.sync_copy(dstr_hbm.at[wid, pl.ds(og * _IGB, _IGB)], dst_v)

        @pl.loop(0, _IGB // 2)
        def _(gidx):
            j0 = gidx * 2
            h0 = pltpu.async_copy(g_hbm.at[src_v.at[j0]], rows0_v, sem0)
            h1 = pltpu.async_copy(g_hbm.at[src_v.at[j0 + 1]], rows1_v, sem1)
            h0.wait()
            h1.wait()
            pltpu.sync_copy(rows0_v, acc_sh.at[dst_v.at[j0]], add=True)
            pltpu.sync_copy(rows1_v, acc_sh.at[dst_v.at[j0 + 1]], add=True)

    plsc.subcore_barrier()
    pltpu.sync_copy(
        acc_sh.at[pl.ds(s * _RPT, _RPT)],
        out_hbm.at[c, pl.ds(s * _RPT, _RPT)],
    )


_RB = 2048


def _dinv_block(degp_blk):
    deg = degp_blk[0, :, 0:1] + degp_blk[1, :, 0:1] + 1.0
    return lax.rsqrt(deg)


def _mm_scale(x, W, degp):
    def body(x_ref, w_ref, degp_ref, o_ref):
        dinv = _dinv_block(degp_ref[...])
        h = jnp.dot(x_ref[...], w_ref[...], preferred_element_type=jnp.float32)
        o_ref[...] = h * dinv

    return pl.pallas_call(
        body,
        grid=(_NP // _RB,),
        in_specs=[
            pl.BlockSpec((_RB, _D), lambda i: (i, 0)),
            pl.BlockSpec((_D, _D), lambda i: (0, 0)),
            pl.BlockSpec((_NC, _RB, 16), lambda i: (0, i, 0)),
        ],
        out_specs=pl.BlockSpec((_RB, _D), lambda i: (i, 0)),
        out_shape=jax.ShapeDtypeStruct((_NP, _D), jnp.float32),
    )(x, W, degp)


def _combine(acc, g, degp, b, W, bias_out, scale_out):
    def body(acc_ref, g_ref, degp_ref, b_ref, w_ref, bo_ref, o_ref):
        dinv = _dinv_block(degp_ref[...])
        tot = (acc_ref[0] + acc_ref[1] + g_ref[...]) * dinv + b_ref[...]
        h = jnp.maximum(tot, 0.0)
        o = jnp.dot(h, w_ref[...], preferred_element_type=jnp.float32)
        if scale_out:
            o = o * dinv
        else:
            o = o + bo_ref[...]
        o_ref[...] = o

    bo = bias_out if bias_out is not None else jnp.zeros((_D,), jnp.float32)
    return pl.pallas_call(
        body,
        grid=(_NP // _RB,),
        in_specs=[
            pl.BlockSpec((_NC, _RB, _D), lambda i: (0, i, 0)),
            pl.BlockSpec((_RB, _D), lambda i: (i, 0)),
            pl.BlockSpec((_NC, _RB, 16), lambda i: (0, i, 0)),
            pl.BlockSpec((1, _D), lambda i: (0, 0)),
            pl.BlockSpec((_D, _D), lambda i: (0, 0)),
            pl.BlockSpec((1, _D), lambda i: (0, 0)),
        ],
        out_specs=pl.BlockSpec((_RB, _D), lambda i: (i, 0)),
        out_shape=jax.ShapeDtypeStruct((_NP, _D), jnp.float32),
    )(acc, g, degp, b.reshape(1, _D), W, bo.reshape(1, _D))


def kernel(x, edge_index, W1, b1, W2, b2, Wl, bl):
    padv = _N + (jnp.arange(_NPAD, dtype=jnp.int32) % (_NP - _N))
    pad = jnp.broadcast_to(padv, (2, _NPAD))
    ei = jnp.concatenate([edge_index, pad], axis=1)
    srcr = ei[0].reshape(_NW, _NCHUNK, _CHUNK)
    dstr = ei[1].reshape(_NW, _NCHUNK, _CHUNK)
    xp = jnp.zeros((_NP, _D), jnp.float32).at[:_N].set(x)
    zeros = jnp.zeros((_NP, _D), jnp.float32)
    zeros16 = jnp.zeros((_NP, 16), jnp.float32)

    degp = _sc_degree(dstr, zeros16)
    g1 = _mm_scale(xp, W1, degp)
    acc1 = _sc_edges(g1, srcr, dstr, zeros)
    g2 = _combine(acc1, g1, degp, b1, W2, None, scale_out=True)
    acc2 = _sc_edges(g2, srcr, dstr, zeros)
    out = _combine(acc2, g2, degp, b2, Wl, bl, scale_out=False)
    return out[:_N]
